# Initial kernel scaffold; baseline (speedup 1.0000x reference)
import jax
import jax.numpy as jnp
from jax.experimental import pallas as pl


def kernel(x, edge_index, edge_attr, We1, be1, W11, b11, W12, b12, We2, be2, W21, b21, W22, b22, Wp1, bp1, Wp2, bp2):
    raise NotImplementedError("write your pallas kernel here")


# speedup vs baseline: 2.7108x; 2.7108x over previous
import dataclasses
import functools

import jax
import jax.numpy as jnp
from jax import lax
from jax.experimental import pallas as pl
from jax.experimental.pallas import tpu as pltpu
from jax.experimental.pallas import tpu_sc as plsc

N = 10000
E = 320000
D = 128
DE = 16
H = 128

NC = 2
NS = 16
NW = NC * NS
EW = E // NW
WIN = 80
NWIN = EW // WIN
IB = 25
NP = 10240
RPT = NP // NS
CH = D // 16

_mesh = plsc.VectorSubcoreMesh(core_axis_name="c", subcore_axis_name="s")

_sc_params = pltpu.CompilerParams()
if "needs_layout_passes" in pltpu.CompilerParams.__dataclass_fields__:
    _sc_params = dataclasses.replace(_sc_params, needs_layout_passes=False)


def _segsum_body(x_hbm, e_hbm, src_hbm, dst_hbm, out_hbm,
                 src_v, dst_v, rows_v, msg_v, acc_sh):
    cid = lax.axis_index("c")
    sid = lax.axis_index("s")
    wid = cid * NS + sid

    @pl.loop(0, WIN)
    def _(r):
        for c in range(CH):
            msg_v.at[r, pl.ds(c * 16, 16)][...] = jnp.zeros((16,), jnp.float32)

    @pl.loop(0, RPT // WIN)
    def _(j):
        pltpu.sync_copy(msg_v, acc_sh.at[pl.ds(sid * RPT + j * WIN, WIN)])

    plsc.subcore_barrier()

    ebase = wid * EW

    @pl.loop(0, NWIN // IB)
    def _(blk):
        pltpu.sync_copy(src_hbm.at[wid, blk], src_v)
        pltpu.sync_copy(dst_hbm.at[wid, blk], dst_v)

        @pl.loop(0, IB)
        def _(ii):
            i = blk * IB + ii
            pltpu.sync_copy(x_hbm.at[src_v.at[ii]], rows_v)
            pltpu.sync_copy(e_hbm.at[pl.ds(ebase + i * WIN, WIN)], msg_v)

            @pl.loop(0, WIN)
            def _(r):
                for c in range(CH):
                    s = (r, pl.ds(c * 16, 16))
                    msg_v.at[*s][...] = jnp.maximum(
                        msg_v.at[*s][...] + rows_v.at[*s][...], 0.0)

            pltpu.sync_copy(msg_v, acc_sh.at[dst_v.at[ii]], add=True)

    plsc.subcore_barrier()
    pltpu.sync_copy(acc_sh.at[pl.ds(sid * RPT, RPT)],
                    out_hbm.at[cid, pl.ds(sid * RPT, RPT)])


_segsum = pl.kernel(
    _segsum_body,
    out_type=jax.ShapeDtypeStruct((NC, NP, D), jnp.float32),
    mesh=_mesh,
    scratch_types=[
        pltpu.VMEM((IB, WIN), jnp.int32),
        pltpu.VMEM((IB, WIN), jnp.int32),
        pltpu.VMEM((WIN, D), jnp.float32),
        pltpu.VMEM((WIN, D), jnp.float32),
        pltpu.VMEM_SHARED((NP, D), jnp.float32),
    ],
    compiler_params=_sc_params,
)


def _pred_body(a_hbm, b_hbm, src_hbm, dst_hbm, wb_hbm, out_hbm,
               src_v, dst_v, a_v, b_v, wb_v, pt_v, out_v):
    cid = lax.axis_index("c")
    sid = lax.axis_index("s")
    wid = cid * NS + sid

    pltpu.sync_copy(src_hbm.at[wid], src_v)
    pltpu.sync_copy(dst_hbm.at[wid], dst_v)
    pltpu.sync_copy(wb_hbm, wb_v)

    ebase = wid * EW
    lanes = lax.iota(jnp.int32, 16)

    @pl.loop(0, NWIN)
    def _(i):
        pltpu.sync_copy(a_hbm.at[src_v.at[i]], a_v)
        pltpu.sync_copy(b_hbm.at[dst_v.at[i]], b_v)

        @pl.loop(0, WIN)
        def _(r):
            acc = jnp.zeros((16,), jnp.float32)
            for c in range(CH):
                s = (r, pl.ds(c * 16, 16))
                m = jnp.maximum(a_v.at[*s][...] + b_v.at[*s][...], 0.0)
                acc = acc + m * wb_v.at[c][...]
            acc = acc + wb_v.at[CH][...]
            plsc.store_scatter(pt_v, [lanes, jnp.full((16,), r, jnp.int32)],
                               acc)

        for g in range(WIN // 16):
            tot = jnp.zeros((16,), jnp.float32)
            for l in range(16):
                tot = tot + pt_v.at[l, pl.ds(g * 16, 16)][...]
            out_v.at[pl.ds(g * 16, 16)][...] = tot

        pltpu.sync_copy(out_v, out_hbm.at[pl.ds(ebase + i * WIN, WIN)])


_pred = pl.kernel(
    _pred_body,
    out_type=jax.ShapeDtypeStruct((E,), jnp.float32),
    mesh=_mesh,
    scratch_types=[
        pltpu.VMEM((NWIN, WIN), jnp.int32),
        pltpu.VMEM((NWIN, WIN), jnp.int32),
        pltpu.VMEM((WIN, D), jnp.float32),
        pltpu.VMEM((WIN, D), jnp.float32),
        pltpu.VMEM((CH + 1, 16), jnp.float32),
        pltpu.VMEM((16, WIN), jnp.float32),
        pltpu.VMEM((WIN,), jnp.float32),
    ],
    compiler_params=_sc_params,
)


BE = 6400


def _eproj_body(attr_ref, we_ref, be_ref, out_ref):
    out_ref[...] = jnp.dot(attr_ref[...], we_ref[...],
                           preferred_element_type=jnp.float32) + be_ref[...]


def _eproj(edge_attr, we, be):
    return pl.pallas_call(
        _eproj_body,
        grid=(E // BE,),
        in_specs=[
            pl.BlockSpec((BE, DE), lambda i: (i, 0)),
            pl.BlockSpec((DE, D), lambda i: (0, 0)),
            pl.BlockSpec((1, D), lambda i: (0, 0)),
        ],
        out_specs=pl.BlockSpec((BE, D), lambda i: (i, 0)),
        out_shape=jax.ShapeDtypeStruct((E, D), jnp.float32),
    )(edge_attr, we, be.reshape(1, D))


BN = 2000


def _node_body(x_ref, p_ref, w1_ref, b1_ref, w2_ref, b2_ref, out_ref):
    p = p_ref[...]
    hin = x_ref[...] + p[0] + p[1]
    t = jnp.maximum(jnp.dot(hin, w1_ref[...],
                            preferred_element_type=jnp.float32) + b1_ref[...],
                    0.0)
    out_ref[...] = jnp.maximum(
        jnp.dot(t, w2_ref[...], preferred_element_type=jnp.float32)
        + b2_ref[...], 0.0)


def _node(x, p, w1, b1, w2, b2):
    return pl.pallas_call(
        _node_body,
        grid=(N // BN,),
        in_specs=[
            pl.BlockSpec((BN, D), lambda i: (i, 0)),
            pl.BlockSpec((NC, BN, D), lambda i: (0, i, 0)),
            pl.BlockSpec((D, H), lambda i: (0, 0)),
            pl.BlockSpec((1, H), lambda i: (0, 0)),
            pl.BlockSpec((H, H), lambda i: (0, 0)),
            pl.BlockSpec((1, H), lambda i: (0, 0)),
        ],
        out_specs=pl.BlockSpec((BN, H), lambda i: (i, 0)),
        out_shape=jax.ShapeDtypeStruct((N, H), jnp.float32),
    )(x, p, w1, b1.reshape(1, H), w2, b2.reshape(1, H))


def _node2_body(x_ref, p_ref, w1_ref, b1_ref, w2_ref, b2_ref, wp1_ref,
                bp1_ref, a_ref, b_out_ref):
    p = p_ref[...]
    hin = x_ref[...] + p[0] + p[1]
    t = jnp.maximum(jnp.dot(hin, w1_ref[...],
                            preferred_element_type=jnp.float32) + b1_ref[...],
                    0.0)
    h2 = jnp.maximum(
        jnp.dot(t, w2_ref[...], preferred_element_type=jnp.float32)
        + b2_ref[...], 0.0)
    wp1 = wp1_ref[...]
    a_ref[...] = jnp.dot(h2, wp1[:H],
                         preferred_element_type=jnp.float32) + bp1_ref[...]
    b_out_ref[...] = jnp.dot(h2, wp1[H:],
                             preferred_element_type=jnp.float32)


def _node2(x, p, w1, b1, w2, b2, wp1, bp1):
    return pl.pallas_call(
        _node2_body,
        grid=(N // BN,),
        in_specs=[
            pl.BlockSpec((BN, D), lambda i: (i, 0)),
            pl.BlockSpec((NC, BN, D), lambda i: (0, i, 0)),
            pl.BlockSpec((D, H), lambda i: (0, 0)),
            pl.BlockSpec((1, H), lambda i: (0, 0)),
            pl.BlockSpec((H, H), lambda i: (0, 0)),
            pl.BlockSpec((1, H), lambda i: (0, 0)),
            pl.BlockSpec((2 * H, H), lambda i: (0, 0)),
            pl.BlockSpec((1, H), lambda i: (0, 0)),
        ],
        out_specs=[
            pl.BlockSpec((BN, H), lambda i: (i, 0)),
            pl.BlockSpec((BN, H), lambda i: (i, 0)),
        ],
        out_shape=[
            jax.ShapeDtypeStruct((N, H), jnp.float32),
            jax.ShapeDtypeStruct((N, H), jnp.float32),
        ],
    )(x, p, w1, b1.reshape(1, H), w2, b2.reshape(1, H), wp1,
      bp1.reshape(1, H))


def kernel(x, edge_index, edge_attr, We1, be1, W11, b11, W12, b12,
           We2, be2, W21, b21, W22, b22, Wp1, bp1, Wp2, bp2):
    src4 = edge_index[0].reshape(NW, NWIN // IB, IB, WIN)
    dst4 = edge_index[1].reshape(NW, NWIN // IB, IB, WIN)
    src = edge_index[0].reshape(NW, NWIN, WIN)
    dst = edge_index[1].reshape(NW, NWIN, WIN)

    e1 = _eproj(edge_attr, We1, be1)
    e2 = _eproj(edge_attr, We2, be2)

    p1 = _segsum(x, e1, src4, dst4)
    h1 = _node(x, p1, W11, b11, W12, b12)

    p2 = _segsum(h1, e2, src4, dst4)
    a, b = _node2(h1, p2, W21, b21, W22, b22, Wp1, bp1)

    wb2 = jnp.concatenate(
        [Wp2[:, 0], bp2, jnp.zeros((15,), jnp.float32)]).reshape(CH + 1, 16)
    return _pred(a, b, src, dst, wb2)
